# baseline (speedup 1.0000x reference)
import functools
import jax
import jax.numpy as jnp
from jax import lax
from jax.experimental import pallas as pl
from jax.experimental.pallas import tpu as pltpu
from jax.experimental.pallas import tpu_sc as plsc

N_NODES = 10000
N_EDGES = 320000
D = 128
N_GRAPHS = 64
N_CLASSES = 10

NC = 2
NS = 16
NW = NC * NS
EPW = N_EDGES // NW
CHUNK = 80
NIT = EPW // CHUNK
NBUF = 4
NPAD = 10112
RPS = NPAD // NS

def _mesh():
    return plsc.VectorSubcoreMesh(core_axis_name="c", subcore_axis_name="s")


def _zero_vmem_2d(ref, rows, cols):
    z = jnp.zeros((16,), jnp.float32)
    def body(r, _):
        for c in range(cols // 16):
            ref[r, pl.ds(c * 16, 16)] = z
        return _
    lax.fori_loop(0, rows, body, None)


def _deg_body(dst3_hbm, out_hbm, d0, d1, d2, ones_v, stage_v, deg_sh, sem_i):
    c = lax.axis_index("c")
    s = lax.axis_index("s")
    w = c * NS + s
    one = jnp.ones((16,), jnp.float32)
    dbufs = (d0, d1, d2)

    _zero_vmem_2d(stage_v, CHUNK, 16)
    for j in range(7):
        pltpu.sync_copy(stage_v, deg_sh.at[pl.ds(s * RPS + j * CHUNK, CHUNK)])
    pltpu.sync_copy(stage_v.at[pl.ds(0, RPS - 7 * CHUNK)],
                    deg_sh.at[pl.ds(s * RPS + 7 * CHUNK, RPS - 7 * CHUNK)])
    def fill(r, _):
        ones_v[r, pl.ds(0, 16)] = one
        return _
    lax.fori_loop(0, CHUNK, fill, None)
    plsc.subcore_barrier()

    pltpu.async_copy(dst3_hbm.at[w, 0], d0, sem_i)
    pltpu.async_copy(dst3_hbm.at[w, 1], d1, sem_i)

    def step(i, _):
        for k in range(3):
            @pl.when(lax.rem(i, 3) == k)
            def _(k=k):
                @pl.when(i + 2 < NIT)
                def _():
                    pltpu.async_copy(dst3_hbm.at[w, i + 2],
                                     dbufs[(k + 2) % 3], sem_i)
                pltpu.make_async_copy(dst3_hbm.at[w, i], dbufs[k],
                                      sem_i).wait()
                pltpu.sync_copy(ones_v, deg_sh.at[dbufs[k]], add=True)
        return _
    lax.fori_loop(0, NIT, step, None)
    plsc.subcore_barrier()

    pltpu.sync_copy(deg_sh.at[pl.ds(s * RPS, RPS)],
                    out_hbm.at[c, pl.ds(s * RPS, RPS)])


def _edge_body(src3_hbm, dst3_hbm, hs_hbm, out_hbm,
               s0, s1, s2, s3, d0, d1, d2, d3, r0, r1, r2, r3,
               acc_sh, sem_g, sem_i, sem_s):
    c = lax.axis_index("c")
    s = lax.axis_index("s")
    w = c * NS + s
    sbufs = (s0, s1, s2, s3)
    dbufs = (d0, d1, d2, d3)
    rows = (r0, r1, r2, r3)

    _zero_vmem_2d(r0, CHUNK, D)
    for j in range(7):
        pltpu.sync_copy(r0, acc_sh.at[pl.ds(s * RPS + j * CHUNK, CHUNK)])
    pltpu.sync_copy(r0.at[pl.ds(0, RPS - 7 * CHUNK)],
                    acc_sh.at[pl.ds(s * RPS + 7 * CHUNK, RPS - 7 * CHUNK)])
    plsc.subcore_barrier()

    for j in range(3):
        pltpu.async_copy(src3_hbm.at[w, j], sbufs[j], sem_i)
        pltpu.async_copy(dst3_hbm.at[w, j], dbufs[j], sem_i)
    pltpu.make_async_copy(src3_hbm.at[w, 0], s0, sem_i).wait()
    pltpu.make_async_copy(dst3_hbm.at[w, 0], d0, sem_i).wait()
    pltpu.async_copy(hs_hbm.at[s0], r0, sem_g)

    def step(i, _):
        for k in range(NBUF):
            @pl.when(lax.rem(i, NBUF) == k)
            def _(k=k):
                kn = (k + 1) % NBUF
                kp = (k + 3) % NBUF
                km = (k + 3) % NBUF
                pltpu.make_async_copy(hs_hbm.at[sbufs[k]], rows[k],
                                      sem_g).wait()
                @pl.when(i >= 1)
                def _():
                    pltpu.make_async_copy(rows[km], acc_sh.at[dbufs[km]],
                                          sem_s).wait()
                @pl.when(i + 3 < NIT)
                def _():
                    pltpu.async_copy(src3_hbm.at[w, i + 3], sbufs[kp], sem_i)
                    pltpu.async_copy(dst3_hbm.at[w, i + 3], dbufs[kp], sem_i)
                @pl.when(i + 1 < NIT)
                def _():
                    pltpu.make_async_copy(src3_hbm.at[w, i + 1], sbufs[kn],
                                          sem_i).wait()
                    pltpu.make_async_copy(dst3_hbm.at[w, i + 1], dbufs[kn],
                                          sem_i).wait()
                    pltpu.async_copy(hs_hbm.at[sbufs[kn]], rows[kn], sem_g)
                pltpu.async_copy(rows[k], acc_sh.at[dbufs[k]], sem_s,
                                 add=True)
        return _
    lax.fori_loop(0, NIT, step, None)
    kl = (NIT - 1) % NBUF
    pltpu.make_async_copy(rows[kl], acc_sh.at[dbufs[kl]], sem_s).wait()
    plsc.subcore_barrier()

    pltpu.sync_copy(acc_sh.at[pl.ds(s * RPS, RPS)],
                    out_hbm.at[c, pl.ds(s * RPS, RPS)])


@functools.cache
def _sc_kernels():
    deg = pl.kernel(
        _deg_body,
        out_type=jax.ShapeDtypeStruct((NC, NPAD, 16), jnp.float32),
        mesh=_mesh(),
        scratch_types=[
            pltpu.VMEM((CHUNK,), jnp.int32),
            pltpu.VMEM((CHUNK,), jnp.int32),
            pltpu.VMEM((CHUNK,), jnp.int32),
            pltpu.VMEM((CHUNK, 16), jnp.float32),
            pltpu.VMEM((CHUNK, 16), jnp.float32),
            pltpu.VMEM_SHARED((NPAD, 16), jnp.float32),
            pltpu.SemaphoreType.DMA,
        ],
    )
    edge = pl.kernel(
        _edge_body,
        out_type=jax.ShapeDtypeStruct((NC, NPAD, D), jnp.float32),
        mesh=_mesh(),
        scratch_types=(
            [pltpu.VMEM((CHUNK,), jnp.int32)] * 8
            + [pltpu.VMEM((CHUNK, D), jnp.float32)] * 4
            + [pltpu.VMEM_SHARED((NPAD, D), jnp.float32),
               pltpu.SemaphoreType.DMA,
               pltpu.SemaphoreType.DMA,
               pltpu.SemaphoreType.DMA]
        ),
    )
    return deg, edge


BLK = 1000
NBLK = N_NODES // BLK
FBLK = 400
NFBLK = N_NODES // FBLK

_dot = functools.partial(jnp.dot, preferred_element_type=jnp.float32,
                         precision=lax.Precision.HIGHEST)


def _tc_pre_body(x_ref, w_ref, d0_ref, d1_ref, hs_ref, dinv_ref):
    deg = jnp.sum(d0_ref[:, :1] + d1_ref[:, :1], axis=1, keepdims=True) + 1.0
    dinv = lax.rsqrt(deg)
    h = _dot(x_ref[...], w_ref[...])
    hs_ref[...] = dinv * h
    dinv_ref[...] = dinv


def _tc_pre(x, W1, dp0, dp1):
    return pl.pallas_call(
        _tc_pre_body,
        grid=(NFBLK,),
        in_specs=[
            pl.BlockSpec((FBLK, D), lambda i: (i, 0)),
            pl.BlockSpec((D, D), lambda i: (0, 0)),
            pl.BlockSpec((FBLK, 16), lambda i: (i, 0)),
            pl.BlockSpec((FBLK, 16), lambda i: (i, 0)),
        ],
        out_specs=[
            pl.BlockSpec((FBLK, D), lambda i: (i, 0)),
            pl.BlockSpec((FBLK, 1), lambda i: (i, 0)),
        ],
        out_shape=[
            jax.ShapeDtypeStruct((N_NODES, D), jnp.float32),
            jax.ShapeDtypeStruct((N_NODES, 1), jnp.float32),
        ],
    )(x, W1, dp0, dp1)


def _tc_mid_body(a0_ref, a1_ref, hs_ref, dinv_ref, b_ref, w_ref, out_ref):
    dinv = dinv_ref[...]
    pre = dinv * (a0_ref[...] + a1_ref[...] + hs_ref[...]) + b_ref[...]
    x2 = jnp.maximum(pre, 0.0)
    out_ref[...] = dinv * _dot(x2, w_ref[...])


def _tc_mid(a0, a1, hs, dinv, b, W):
    return pl.pallas_call(
        _tc_mid_body,
        grid=(NBLK,),
        in_specs=[
            pl.BlockSpec((BLK, D), lambda i: (i, 0)),
            pl.BlockSpec((BLK, D), lambda i: (i, 0)),
            pl.BlockSpec((BLK, D), lambda i: (i, 0)),
            pl.BlockSpec((BLK, 1), lambda i: (i, 0)),
            pl.BlockSpec((1, D), lambda i: (0, 0)),
            pl.BlockSpec((D, D), lambda i: (0, 0)),
        ],
        out_specs=pl.BlockSpec((BLK, D), lambda i: (i, 0)),
        out_shape=jax.ShapeDtypeStruct((N_NODES, D), jnp.float32),
    )(a0, a1, hs, dinv, b, W)


def _tc_fin_body(a0_ref, a1_ref, hs_ref, dinv_ref, b_ref, brow_ref, bcol_ref,
                 wl_ref, bl_ref, out_ref, sum_ref, max_ref, cnt_ref):
    i = pl.program_id(0)

    @pl.when(i == 0)
    def _():
        sum_ref[...] = jnp.zeros((N_GRAPHS, D), jnp.float32)
        max_ref[...] = jnp.full((N_GRAPHS, D), -jnp.inf, jnp.float32)
        cnt_ref[...] = jnp.zeros((N_GRAPHS, 1), jnp.float32)

    h = dinv_ref[...] * (a0_ref[...] + a1_ref[...] + hs_ref[...]) + b_ref[...]

    ids_row = brow_ref[0]
    ids_col = bcol_ref[...]
    gi = lax.broadcasted_iota(jnp.int32, (N_GRAPHS, FBLK), 0)
    onehot = (ids_row == gi).astype(jnp.float32)

    sum_blk = _dot(onehot, h)
    cnt_blk = jnp.sum(onehot, axis=1, keepdims=True)

    v = h
    k = 1
    while k < FBLK:
        pv = jnp.concatenate(
            [jnp.full((k, D), -jnp.inf, jnp.float32), v[:-k]], axis=0)
        pid = jnp.concatenate(
            [jnp.full((k, 1), -1, jnp.int32), ids_col[:-k]], axis=0)
        v = jnp.where(pid == ids_col, jnp.maximum(v, pv), v)
        k *= 2
    nxt = jnp.concatenate(
        [ids_row[:, 1:], jnp.full((1, 1), -1, jnp.int32)], axis=1)
    last = (ids_row != nxt).astype(jnp.float32)
    maxseg = _dot(onehot * last, v)
    maxseg = jnp.where(cnt_blk > 0, maxseg, -jnp.inf)

    sum_ref[...] += sum_blk
    cnt_ref[...] += cnt_blk
    max_ref[...] = jnp.maximum(max_ref[...], maxseg)

    @pl.when(i == NFBLK - 1)
    def _():
        cnt = cnt_ref[...]
        mean = sum_ref[...] / jnp.maximum(cnt, 1.0)
        mx = jnp.where(cnt > 0, max_ref[...], 0.0)
        pooled = jnp.concatenate([mean, mx], axis=1)
        out_ref[...] = _dot(pooled, wl_ref[...]) + bl_ref[...]


def _tc_fin(a0, a1, hs, dinv, b, brow, bcol, Wl, bl):
    return pl.pallas_call(
        _tc_fin_body,
        grid=(NFBLK,),
        in_specs=[
            pl.BlockSpec((FBLK, D), lambda i: (i, 0)),
            pl.BlockSpec((FBLK, D), lambda i: (i, 0)),
            pl.BlockSpec((FBLK, D), lambda i: (i, 0)),
            pl.BlockSpec((FBLK, 1), lambda i: (i, 0)),
            pl.BlockSpec((1, D), lambda i: (0, 0)),
            pl.BlockSpec((1, 1, FBLK), lambda i: (i, 0, 0)),
            pl.BlockSpec((FBLK, 1), lambda i: (i, 0)),
            pl.BlockSpec((2 * D, N_CLASSES), lambda i: (0, 0)),
            pl.BlockSpec((1, N_CLASSES), lambda i: (0, 0)),
        ],
        out_specs=pl.BlockSpec((N_GRAPHS, N_CLASSES), lambda i: (0, 0)),
        out_shape=jax.ShapeDtypeStruct((N_GRAPHS, N_CLASSES), jnp.float32),
        scratch_shapes=[
            pltpu.VMEM((N_GRAPHS, D), jnp.float32),
            pltpu.VMEM((N_GRAPHS, D), jnp.float32),
            pltpu.VMEM((N_GRAPHS, 1), jnp.float32),
        ],
    )(a0, a1, hs, dinv, b, brow, bcol, Wl, bl)


def kernel(x, edge_index, batch, W1, b1, W2, b2, W3, b3, Wl, bl):
    src3 = edge_index[0].reshape(NW, NIT, CHUNK)
    dst3 = edge_index[1].reshape(NW, NIT, CHUNK)
    _deg_sc, _edge_sc = _sc_kernels()

    degp = _deg_sc(dst3)
    hs1, dinv = _tc_pre(x, W1, degp[0], degp[1])
    acc1 = _edge_sc(src3, dst3, hs1)
    hs2 = _tc_mid(acc1[0], acc1[1], hs1, dinv, b1.reshape(1, D), W2)
    acc2 = _edge_sc(src3, dst3, hs2)
    hs3 = _tc_mid(acc2[0], acc2[1], hs2, dinv, b2.reshape(1, D), W3)
    acc3 = _edge_sc(src3, dst3, hs3)
    out = _tc_fin(acc3[0], acc3[1], hs3, dinv, b3.reshape(1, D),
                  batch.reshape(NFBLK, 1, FBLK), batch.reshape(N_NODES, 1),
                  Wl, bl.reshape(1, N_CLASSES))
    return out


# speedup vs baseline: 1.0033x; 1.0033x over previous
import functools
import jax
import jax.numpy as jnp
from jax import lax
from jax.experimental import pallas as pl
from jax.experimental.pallas import tpu as pltpu
from jax.experimental.pallas import tpu_sc as plsc

N_NODES = 10000
N_EDGES = 320000
D = 128
N_GRAPHS = 64
N_CLASSES = 10

NC = 2
NS = 16
NW = NC * NS
EPW = N_EDGES // NW
CHUNK = 80
NIT = EPW // CHUNK
NBUF = 4
NPAD = 10112
RPS = NPAD // NS

def _mesh():
    return plsc.VectorSubcoreMesh(core_axis_name="c", subcore_axis_name="s")


def _zero_vmem_2d(ref, rows, cols):
    z = jnp.zeros((16,), jnp.float32)
    def body(r, _):
        for c in range(cols // 16):
            ref[r, pl.ds(c * 16, 16)] = z
        return _
    lax.fori_loop(0, rows, body, None)


def _deg_body(dst3_hbm, out_hbm, d0, d1, d2, ones_v, stage_v, deg_sh, sem_i):
    c = lax.axis_index("c")
    s = lax.axis_index("s")
    w = c * NS + s
    one = jnp.ones((16,), jnp.float32)
    dbufs = (d0, d1, d2)

    _zero_vmem_2d(stage_v, CHUNK, 16)
    for j in range(7):
        pltpu.sync_copy(stage_v, deg_sh.at[pl.ds(s * RPS + j * CHUNK, CHUNK)])
    pltpu.sync_copy(stage_v.at[pl.ds(0, RPS - 7 * CHUNK)],
                    deg_sh.at[pl.ds(s * RPS + 7 * CHUNK, RPS - 7 * CHUNK)])
    def fill(r, _):
        ones_v[r, pl.ds(0, 16)] = one
        return _
    lax.fori_loop(0, CHUNK, fill, None)
    plsc.subcore_barrier()

    pltpu.async_copy(dst3_hbm.at[w, 0], d0, sem_i)
    pltpu.async_copy(dst3_hbm.at[w, 1], d1, sem_i)

    def step(i, _):
        for k in range(3):
            @pl.when(lax.rem(i, 3) == k)
            def _(k=k):
                @pl.when(i + 2 < NIT)
                def _():
                    pltpu.async_copy(dst3_hbm.at[w, i + 2],
                                     dbufs[(k + 2) % 3], sem_i)
                pltpu.make_async_copy(dst3_hbm.at[w, i], dbufs[k],
                                      sem_i).wait()
                pltpu.sync_copy(ones_v, deg_sh.at[dbufs[k]], add=True)
        return _
    lax.fori_loop(0, NIT, step, None)
    plsc.subcore_barrier()

    pltpu.sync_copy(deg_sh.at[pl.ds(s * RPS, RPS)],
                    out_hbm.at[c, pl.ds(s * RPS, RPS)])


def _edge_body(src3_hbm, dst3_hbm, hs_hbm, out_hbm,
               s0, s1, s2, s3, d0, d1, d2, d3, r0, r1, r2, r3,
               acc_sh, sem_g, sem_i, sem_s):
    c = lax.axis_index("c")
    s = lax.axis_index("s")
    w = c * NS + s
    sbufs = (s0, s1, s2, s3)
    dbufs = (d0, d1, d2, d3)
    rows = (r0, r1, r2, r3)

    _zero_vmem_2d(r0, CHUNK, D)
    for j in range(7):
        pltpu.sync_copy(r0, acc_sh.at[pl.ds(s * RPS + j * CHUNK, CHUNK)])
    pltpu.sync_copy(r0.at[pl.ds(0, RPS - 7 * CHUNK)],
                    acc_sh.at[pl.ds(s * RPS + 7 * CHUNK, RPS - 7 * CHUNK)])
    plsc.subcore_barrier()

    for j in range(3):
        pltpu.async_copy(src3_hbm.at[w, j], sbufs[j], sem_i)
        pltpu.async_copy(dst3_hbm.at[w, j], dbufs[j], sem_i)
    pltpu.make_async_copy(src3_hbm.at[w, 0], s0, sem_i).wait()
    pltpu.make_async_copy(dst3_hbm.at[w, 0], d0, sem_i).wait()
    pltpu.async_copy(hs_hbm.at[s0.at[pl.ds(0, 40)]], r0.at[pl.ds(0, 40)],
                     sem_g)
    pltpu.async_copy(hs_hbm.at[s0.at[pl.ds(40, 40)]], r0.at[pl.ds(40, 40)],
                     sem_g)

    def step(i, _):
        for k in range(NBUF):
            @pl.when(lax.rem(i, NBUF) == k)
            def _(k=k):
                kn = (k + 1) % NBUF
                kp = (k + 3) % NBUF
                km = (k + 3) % NBUF
                pltpu.make_async_copy(hs_hbm.at[sbufs[k].at[pl.ds(0, 40)]],
                                      rows[k].at[pl.ds(0, 40)], sem_g).wait()
                pltpu.make_async_copy(hs_hbm.at[sbufs[k].at[pl.ds(40, 40)]],
                                      rows[k].at[pl.ds(40, 40)], sem_g).wait()
                @pl.when(i >= 1)
                def _():
                    pltpu.make_async_copy(rows[km], acc_sh.at[dbufs[km]],
                                          sem_s).wait()
                @pl.when(i + 3 < NIT)
                def _():
                    pltpu.async_copy(src3_hbm.at[w, i + 3], sbufs[kp], sem_i)
                    pltpu.async_copy(dst3_hbm.at[w, i + 3], dbufs[kp], sem_i)
                @pl.when(i + 1 < NIT)
                def _():
                    pltpu.make_async_copy(src3_hbm.at[w, i + 1], sbufs[kn],
                                          sem_i).wait()
                    pltpu.make_async_copy(dst3_hbm.at[w, i + 1], dbufs[kn],
                                          sem_i).wait()
                    pltpu.async_copy(hs_hbm.at[sbufs[kn].at[pl.ds(0, 40)]],
                                     rows[kn].at[pl.ds(0, 40)], sem_g)
                    pltpu.async_copy(hs_hbm.at[sbufs[kn].at[pl.ds(40, 40)]],
                                     rows[kn].at[pl.ds(40, 40)], sem_g)
                pltpu.async_copy(rows[k], acc_sh.at[dbufs[k]], sem_s,
                                 add=True)
        return _
    lax.fori_loop(0, NIT, step, None)
    kl = (NIT - 1) % NBUF
    pltpu.make_async_copy(rows[kl], acc_sh.at[dbufs[kl]], sem_s).wait()
    plsc.subcore_barrier()

    pltpu.sync_copy(acc_sh.at[pl.ds(s * RPS, RPS)],
                    out_hbm.at[c, pl.ds(s * RPS, RPS)])


@functools.cache
def _sc_kernels():
    deg = pl.kernel(
        _deg_body,
        out_type=jax.ShapeDtypeStruct((NC, NPAD, 16), jnp.float32),
        mesh=_mesh(),
        scratch_types=[
            pltpu.VMEM((CHUNK,), jnp.int32),
            pltpu.VMEM((CHUNK,), jnp.int32),
            pltpu.VMEM((CHUNK,), jnp.int32),
            pltpu.VMEM((CHUNK, 16), jnp.float32),
            pltpu.VMEM((CHUNK, 16), jnp.float32),
            pltpu.VMEM_SHARED((NPAD, 16), jnp.float32),
            pltpu.SemaphoreType.DMA,
        ],
    )
    edge = pl.kernel(
        _edge_body,
        out_type=jax.ShapeDtypeStruct((NC, NPAD, D), jnp.float32),
        mesh=_mesh(),
        scratch_types=(
            [pltpu.VMEM((CHUNK,), jnp.int32)] * 8
            + [pltpu.VMEM((CHUNK, D), jnp.float32)] * 4
            + [pltpu.VMEM_SHARED((NPAD, D), jnp.float32),
               pltpu.SemaphoreType.DMA,
               pltpu.SemaphoreType.DMA,
               pltpu.SemaphoreType.DMA]
        ),
    )
    return deg, edge


BLK = 1000
NBLK = N_NODES // BLK
FBLK = 400
NFBLK = N_NODES // FBLK

_dot = functools.partial(jnp.dot, preferred_element_type=jnp.float32,
                         precision=lax.Precision.HIGHEST)


def _tc_pre_body(x_ref, w_ref, d0_ref, d1_ref, hs_ref, dinv_ref):
    deg = jnp.sum(d0_ref[:, :1] + d1_ref[:, :1], axis=1, keepdims=True) + 1.0
    dinv = lax.rsqrt(deg)
    h = _dot(x_ref[...], w_ref[...])
    hs_ref[...] = dinv * h
    dinv_ref[...] = dinv


def _tc_pre(x, W1, dp0, dp1):
    return pl.pallas_call(
        _tc_pre_body,
        grid=(NFBLK,),
        in_specs=[
            pl.BlockSpec((FBLK, D), lambda i: (i, 0)),
            pl.BlockSpec((D, D), lambda i: (0, 0)),
            pl.BlockSpec((FBLK, 16), lambda i: (i, 0)),
            pl.BlockSpec((FBLK, 16), lambda i: (i, 0)),
        ],
        out_specs=[
            pl.BlockSpec((FBLK, D), lambda i: (i, 0)),
            pl.BlockSpec((FBLK, 1), lambda i: (i, 0)),
        ],
        out_shape=[
            jax.ShapeDtypeStruct((N_NODES, D), jnp.float32),
            jax.ShapeDtypeStruct((N_NODES, 1), jnp.float32),
        ],
    )(x, W1, dp0, dp1)


def _tc_mid_body(a0_ref, a1_ref, hs_ref, dinv_ref, b_ref, w_ref, out_ref):
    dinv = dinv_ref[...]
    pre = dinv * (a0_ref[...] + a1_ref[...] + hs_ref[...]) + b_ref[...]
    x2 = jnp.maximum(pre, 0.0)
    out_ref[...] = dinv * _dot(x2, w_ref[...])


def _tc_mid(a0, a1, hs, dinv, b, W):
    return pl.pallas_call(
        _tc_mid_body,
        grid=(NBLK,),
        in_specs=[
            pl.BlockSpec((BLK, D), lambda i: (i, 0)),
            pl.BlockSpec((BLK, D), lambda i: (i, 0)),
            pl.BlockSpec((BLK, D), lambda i: (i, 0)),
            pl.BlockSpec((BLK, 1), lambda i: (i, 0)),
            pl.BlockSpec((1, D), lambda i: (0, 0)),
            pl.BlockSpec((D, D), lambda i: (0, 0)),
        ],
        out_specs=pl.BlockSpec((BLK, D), lambda i: (i, 0)),
        out_shape=jax.ShapeDtypeStruct((N_NODES, D), jnp.float32),
    )(a0, a1, hs, dinv, b, W)


def _tc_fin_body(a0_ref, a1_ref, hs_ref, dinv_ref, b_ref, brow_ref, bcol_ref,
                 wl_ref, bl_ref, out_ref, sum_ref, max_ref, cnt_ref):
    i = pl.program_id(0)

    @pl.when(i == 0)
    def _():
        sum_ref[...] = jnp.zeros((N_GRAPHS, D), jnp.float32)
        max_ref[...] = jnp.full((N_GRAPHS, D), -jnp.inf, jnp.float32)
        cnt_ref[...] = jnp.zeros((N_GRAPHS, 1), jnp.float32)

    h = dinv_ref[...] * (a0_ref[...] + a1_ref[...] + hs_ref[...]) + b_ref[...]

    ids_row = brow_ref[0]
    ids_col = bcol_ref[...]
    gi = lax.broadcasted_iota(jnp.int32, (N_GRAPHS, FBLK), 0)
    onehot = (ids_row == gi).astype(jnp.float32)

    sum_blk = _dot(onehot, h)
    cnt_blk = jnp.sum(onehot, axis=1, keepdims=True)

    v = h
    k = 1
    while k < FBLK:
        pv = jnp.concatenate(
            [jnp.full((k, D), -jnp.inf, jnp.float32), v[:-k]], axis=0)
        pid = jnp.concatenate(
            [jnp.full((k, 1), -1, jnp.int32), ids_col[:-k]], axis=0)
        v = jnp.where(pid == ids_col, jnp.maximum(v, pv), v)
        k *= 2
    nxt = jnp.concatenate(
        [ids_row[:, 1:], jnp.full((1, 1), -1, jnp.int32)], axis=1)
    last = (ids_row != nxt).astype(jnp.float32)
    maxseg = _dot(onehot * last, v)
    maxseg = jnp.where(cnt_blk > 0, maxseg, -jnp.inf)

    sum_ref[...] += sum_blk
    cnt_ref[...] += cnt_blk
    max_ref[...] = jnp.maximum(max_ref[...], maxseg)

    @pl.when(i == NFBLK - 1)
    def _():
        cnt = cnt_ref[...]
        mean = sum_ref[...] / jnp.maximum(cnt, 1.0)
        mx = jnp.where(cnt > 0, max_ref[...], 0.0)
        pooled = jnp.concatenate([mean, mx], axis=1)
        out_ref[...] = _dot(pooled, wl_ref[...]) + bl_ref[...]


def _tc_fin(a0, a1, hs, dinv, b, brow, bcol, Wl, bl):
    return pl.pallas_call(
        _tc_fin_body,
        grid=(NFBLK,),
        in_specs=[
            pl.BlockSpec((FBLK, D), lambda i: (i, 0)),
            pl.BlockSpec((FBLK, D), lambda i: (i, 0)),
            pl.BlockSpec((FBLK, D), lambda i: (i, 0)),
            pl.BlockSpec((FBLK, 1), lambda i: (i, 0)),
            pl.BlockSpec((1, D), lambda i: (0, 0)),
            pl.BlockSpec((1, 1, FBLK), lambda i: (i, 0, 0)),
            pl.BlockSpec((FBLK, 1), lambda i: (i, 0)),
            pl.BlockSpec((2 * D, N_CLASSES), lambda i: (0, 0)),
            pl.BlockSpec((1, N_CLASSES), lambda i: (0, 0)),
        ],
        out_specs=pl.BlockSpec((N_GRAPHS, N_CLASSES), lambda i: (0, 0)),
        out_shape=jax.ShapeDtypeStruct((N_GRAPHS, N_CLASSES), jnp.float32),
        scratch_shapes=[
            pltpu.VMEM((N_GRAPHS, D), jnp.float32),
            pltpu.VMEM((N_GRAPHS, D), jnp.float32),
            pltpu.VMEM((N_GRAPHS, 1), jnp.float32),
        ],
    )(a0, a1, hs, dinv, b, brow, bcol, Wl, bl)


def kernel(x, edge_index, batch, W1, b1, W2, b2, W3, b3, Wl, bl):
    src3 = edge_index[0].reshape(NW, NIT, CHUNK)
    dst3 = edge_index[1].reshape(NW, NIT, CHUNK)
    _deg_sc, _edge_sc = _sc_kernels()

    degp = _deg_sc(dst3)
    hs1, dinv = _tc_pre(x, W1, degp[0], degp[1])
    acc1 = _edge_sc(src3, dst3, hs1)
    hs2 = _tc_mid(acc1[0], acc1[1], hs1, dinv, b1.reshape(1, D), W2)
    acc2 = _edge_sc(src3, dst3, hs2)
    hs3 = _tc_mid(acc2[0], acc2[1], hs2, dinv, b2.reshape(1, D), W3)
    acc3 = _edge_sc(src3, dst3, hs3)
    out = _tc_fin(acc3[0], acc3[1], hs3, dinv, b3.reshape(1, D),
                  batch.reshape(NFBLK, 1, FBLK), batch.reshape(N_NODES, 1),
                  Wl, bl.reshape(1, N_CLASSES))
    return out


# speedup vs baseline: 1.0489x; 1.0454x over previous
import functools
import jax
import jax.numpy as jnp
from jax import lax
from jax.experimental import pallas as pl
from jax.experimental.pallas import tpu as pltpu
from jax.experimental.pallas import tpu_sc as plsc

N_NODES = 10000
N_EDGES = 320000
D = 128
N_GRAPHS = 64
N_CLASSES = 10

NC = 2
NS = 16
NW = NC * NS
EPW = N_EDGES // NW
CHUNK = 80
NIT = EPW // CHUNK
NBUF = 4
NPAD = 10112
RPS = NPAD // NS

def _mesh():
    return plsc.VectorSubcoreMesh(core_axis_name="c", subcore_axis_name="s")


def _zero_vmem_2d(ref, rows, cols):
    z = jnp.zeros((16,), jnp.float32)
    def body(r, _):
        for c in range(cols // 16):
            ref[r, pl.ds(c * 16, 16)] = z
        return _
    lax.fori_loop(0, rows, body, None)


def _deg_body(dst3_hbm, out_hbm, d0, d1, d2, ones_v, stage_v, deg_sh, sem_i):
    c = lax.axis_index("c")
    s = lax.axis_index("s")
    w = c * NS + s
    one = jnp.ones((16,), jnp.float32)
    dbufs = (d0, d1, d2)

    _zero_vmem_2d(stage_v, CHUNK, 16)
    for j in range(7):
        pltpu.sync_copy(stage_v, deg_sh.at[pl.ds(s * RPS + j * CHUNK, CHUNK)])
    pltpu.sync_copy(stage_v.at[pl.ds(0, RPS - 7 * CHUNK)],
                    deg_sh.at[pl.ds(s * RPS + 7 * CHUNK, RPS - 7 * CHUNK)])
    def fill(r, _):
        ones_v[r, pl.ds(0, 16)] = one
        return _
    lax.fori_loop(0, CHUNK, fill, None)
    plsc.subcore_barrier()

    pltpu.async_copy(dst3_hbm.at[w, 0], d0, sem_i)
    pltpu.async_copy(dst3_hbm.at[w, 1], d1, sem_i)

    def step(i, _):
        for k in range(3):
            @pl.when(lax.rem(i, 3) == k)
            def _(k=k):
                @pl.when(i + 2 < NIT)
                def _():
                    pltpu.async_copy(dst3_hbm.at[w, i + 2],
                                     dbufs[(k + 2) % 3], sem_i)
                pltpu.make_async_copy(dst3_hbm.at[w, i], dbufs[k],
                                      sem_i).wait()
                pltpu.sync_copy(ones_v, deg_sh.at[dbufs[k]], add=True)
        return _
    lax.fori_loop(0, NIT, step, None)
    plsc.subcore_barrier()

    pltpu.sync_copy(deg_sh.at[pl.ds(s * RPS, RPS)],
                    out_hbm.at[c, pl.ds(s * RPS, RPS)])


def _edge_body(src3_hbm, dst3_hbm, hs_hbm, out_hbm,
               s0, s1, s2, s3, d0, d1, d2, d3, r0, r1, r2, r3,
               acc_sh, sem_g, sem_i, sem_s):
    c = lax.axis_index("c")
    s = lax.axis_index("s")
    w = c * NS + s
    sbufs = (s0, s1, s2, s3)
    dbufs = (d0, d1, d2, d3)
    rows = (r0, r1, r2, r3)

    _zero_vmem_2d(r0, CHUNK, D)
    for j in range(7):
        pltpu.sync_copy(r0, acc_sh.at[pl.ds(s * RPS + j * CHUNK, CHUNK)])
    pltpu.sync_copy(r0.at[pl.ds(0, RPS - 7 * CHUNK)],
                    acc_sh.at[pl.ds(s * RPS + 7 * CHUNK, RPS - 7 * CHUNK)])
    plsc.subcore_barrier()

    for j in range(3):
        pltpu.async_copy(src3_hbm.at[w, j], sbufs[j], sem_i)
        pltpu.async_copy(dst3_hbm.at[w, j], dbufs[j], sem_i)
    pltpu.make_async_copy(src3_hbm.at[w, 0], s0, sem_i).wait()
    pltpu.make_async_copy(dst3_hbm.at[w, 0], d0, sem_i).wait()
    pltpu.async_copy(hs_hbm.at[s0], r0, sem_g)

    def step(i, _):
        for k in range(NBUF):
            @pl.when(lax.rem(i, NBUF) == k)
            def _(k=k):
                kn = (k + 1) % NBUF
                kp = (k + 3) % NBUF
                km = (k + 3) % NBUF
                pltpu.make_async_copy(hs_hbm.at[sbufs[k]], rows[k],
                                      sem_g).wait()
                @pl.when(i >= 1)
                def _():
                    pltpu.make_async_copy(rows[km], acc_sh.at[dbufs[km]],
                                          sem_s).wait()
                @pl.when(i + 3 < NIT)
                def _():
                    pltpu.async_copy(src3_hbm.at[w, i + 3], sbufs[kp], sem_i)
                    pltpu.async_copy(dst3_hbm.at[w, i + 3], dbufs[kp], sem_i)
                @pl.when(i + 1 < NIT)
                def _():
                    pltpu.make_async_copy(src3_hbm.at[w, i + 1], sbufs[kn],
                                          sem_i).wait()
                    pltpu.make_async_copy(dst3_hbm.at[w, i + 1], dbufs[kn],
                                          sem_i).wait()
                    pltpu.async_copy(hs_hbm.at[sbufs[kn]], rows[kn], sem_g)
                pltpu.async_copy(rows[k], acc_sh.at[dbufs[k]], sem_s,
                                 add=True)
        return _
    lax.fori_loop(0, NIT, step, None)
    kl = (NIT - 1) % NBUF
    pltpu.make_async_copy(rows[kl], acc_sh.at[dbufs[kl]], sem_s).wait()
    plsc.subcore_barrier()

    pltpu.sync_copy(acc_sh.at[pl.ds(s * RPS, RPS)],
                    out_hbm.at[c, pl.ds(s * RPS, RPS)])


@functools.cache
def _sc_kernels():
    deg = pl.kernel(
        _deg_body,
        out_type=jax.ShapeDtypeStruct((NC, NPAD, 16), jnp.float32),
        mesh=_mesh(),
        scratch_types=[
            pltpu.VMEM((CHUNK,), jnp.int32),
            pltpu.VMEM((CHUNK,), jnp.int32),
            pltpu.VMEM((CHUNK,), jnp.int32),
            pltpu.VMEM((CHUNK, 16), jnp.float32),
            pltpu.VMEM((CHUNK, 16), jnp.float32),
            pltpu.VMEM_SHARED((NPAD, 16), jnp.float32),
            pltpu.SemaphoreType.DMA,
        ],
    )
    edge = pl.kernel(
        _edge_body,
        out_type=jax.ShapeDtypeStruct((NC, NPAD, D), jnp.float32),
        mesh=_mesh(),
        scratch_types=(
            [pltpu.VMEM((CHUNK,), jnp.int32)] * 8
            + [pltpu.VMEM((CHUNK, D), jnp.float32)] * 4
            + [pltpu.VMEM_SHARED((NPAD, D), jnp.float32),
               pltpu.SemaphoreType.DMA,
               pltpu.SemaphoreType.DMA,
               pltpu.SemaphoreType.DMA]
        ),
    )
    return deg, edge


BLK = 1000
NBLK = N_NODES // BLK
FBLK = 400
NFBLK = N_NODES // FBLK

_dot = functools.partial(jnp.dot, preferred_element_type=jnp.float32,
                         precision=lax.Precision.HIGHEST)


def _tc_pre_body(x_ref, w_ref, d0_ref, d1_ref, hs_ref, dinv_ref):
    deg = jnp.sum(d0_ref[0][:, :1] + d1_ref[0][:, :1], axis=1,
                  keepdims=True) + 1.0
    dinv = lax.rsqrt(deg)
    h = _dot(x_ref[...], w_ref[...])
    hs_ref[...] = dinv * h
    dinv_ref[...] = dinv


def _tc_pre(x, W1, dp):
    return pl.pallas_call(
        _tc_pre_body,
        grid=(NFBLK,),
        in_specs=[
            pl.BlockSpec((FBLK, D), lambda i: (i, 0)),
            pl.BlockSpec((D, D), lambda i: (0, 0)),
            pl.BlockSpec((1, FBLK, 16), lambda i: (0, i, 0)),
            pl.BlockSpec((1, FBLK, 16), lambda i: (1, i, 0)),
        ],
        out_specs=[
            pl.BlockSpec((FBLK, D), lambda i: (i, 0)),
            pl.BlockSpec((FBLK, 1), lambda i: (i, 0)),
        ],
        out_shape=[
            jax.ShapeDtypeStruct((N_NODES, D), jnp.float32),
            jax.ShapeDtypeStruct((N_NODES, 1), jnp.float32),
        ],
    )(x, W1, dp, dp)


def _tc_mid_body(a0_ref, a1_ref, hs_ref, dinv_ref, b_ref, w_ref, out_ref):
    dinv = dinv_ref[...]
    pre = dinv * (a0_ref[0] + a1_ref[0] + hs_ref[...]) + b_ref[...]
    x2 = jnp.maximum(pre, 0.0)
    out_ref[...] = dinv * _dot(x2, w_ref[...])


def _tc_mid(acc, hs, dinv, b, W):
    return pl.pallas_call(
        _tc_mid_body,
        grid=(NBLK,),
        in_specs=[
            pl.BlockSpec((1, BLK, D), lambda i: (0, i, 0)),
            pl.BlockSpec((1, BLK, D), lambda i: (1, i, 0)),
            pl.BlockSpec((BLK, D), lambda i: (i, 0)),
            pl.BlockSpec((BLK, 1), lambda i: (i, 0)),
            pl.BlockSpec((1, D), lambda i: (0, 0)),
            pl.BlockSpec((D, D), lambda i: (0, 0)),
        ],
        out_specs=pl.BlockSpec((BLK, D), lambda i: (i, 0)),
        out_shape=jax.ShapeDtypeStruct((N_NODES, D), jnp.float32),
    )(acc, acc, hs, dinv, b, W)


def _tc_fin_body(a0_ref, a1_ref, hs_ref, dinv_ref, b_ref, brow_ref, bcol_ref,
                 wl_ref, bl_ref, out_ref, sum_ref, max_ref, cnt_ref):
    i = pl.program_id(0)

    @pl.when(i == 0)
    def _():
        sum_ref[...] = jnp.zeros((N_GRAPHS, D), jnp.float32)
        max_ref[...] = jnp.full((N_GRAPHS, D), -jnp.inf, jnp.float32)
        cnt_ref[...] = jnp.zeros((N_GRAPHS, 1), jnp.float32)

    h = dinv_ref[...] * (a0_ref[0] + a1_ref[0] + hs_ref[...]) + b_ref[...]

    ids_row = brow_ref[0]
    ids_col = bcol_ref[...]
    gi = lax.broadcasted_iota(jnp.int32, (N_GRAPHS, FBLK), 0)
    onehot = (ids_row == gi).astype(jnp.float32)

    sum_blk = _dot(onehot, h)
    cnt_blk = jnp.sum(onehot, axis=1, keepdims=True)

    v = h
    k = 1
    while k < FBLK:
        pv = jnp.concatenate(
            [jnp.full((k, D), -jnp.inf, jnp.float32), v[:-k]], axis=0)
        pid = jnp.concatenate(
            [jnp.full((k, 1), -1, jnp.int32), ids_col[:-k]], axis=0)
        v = jnp.where(pid == ids_col, jnp.maximum(v, pv), v)
        k *= 2
    nxt = jnp.concatenate(
        [ids_row[:, 1:], jnp.full((1, 1), -1, jnp.int32)], axis=1)
    last = (ids_row != nxt).astype(jnp.float32)
    maxseg = _dot(onehot * last, v)
    maxseg = jnp.where(cnt_blk > 0, maxseg, -jnp.inf)

    sum_ref[...] += sum_blk
    cnt_ref[...] += cnt_blk
    max_ref[...] = jnp.maximum(max_ref[...], maxseg)

    @pl.when(i == NFBLK - 1)
    def _():
        cnt = cnt_ref[...]
        mean = sum_ref[...] / jnp.maximum(cnt, 1.0)
        mx = jnp.where(cnt > 0, max_ref[...], 0.0)
        pooled = jnp.concatenate([mean, mx], axis=1)
        out_ref[...] = _dot(pooled, wl_ref[...]) + bl_ref[...]


def _tc_fin(acc, hs, dinv, b, brow, bcol, Wl, bl):
    return pl.pallas_call(
        _tc_fin_body,
        grid=(NFBLK,),
        in_specs=[
            pl.BlockSpec((1, FBLK, D), lambda i: (0, i, 0)),
            pl.BlockSpec((1, FBLK, D), lambda i: (1, i, 0)),
            pl.BlockSpec((FBLK, D), lambda i: (i, 0)),
            pl.BlockSpec((FBLK, 1), lambda i: (i, 0)),
            pl.BlockSpec((1, D), lambda i: (0, 0)),
            pl.BlockSpec((1, 1, FBLK), lambda i: (i, 0, 0)),
            pl.BlockSpec((FBLK, 1), lambda i: (i, 0)),
            pl.BlockSpec((2 * D, N_CLASSES), lambda i: (0, 0)),
            pl.BlockSpec((1, N_CLASSES), lambda i: (0, 0)),
        ],
        out_specs=pl.BlockSpec((N_GRAPHS, N_CLASSES), lambda i: (0, 0)),
        out_shape=jax.ShapeDtypeStruct((N_GRAPHS, N_CLASSES), jnp.float32),
        scratch_shapes=[
            pltpu.VMEM((N_GRAPHS, D), jnp.float32),
            pltpu.VMEM((N_GRAPHS, D), jnp.float32),
            pltpu.VMEM((N_GRAPHS, 1), jnp.float32),
        ],
    )(acc, acc, hs, dinv, b, brow, bcol, Wl, bl)


def kernel(x, edge_index, batch, W1, b1, W2, b2, W3, b3, Wl, bl):
    src3 = edge_index[0].reshape(NW, NIT, CHUNK)
    dst3 = edge_index[1].reshape(NW, NIT, CHUNK)
    _deg_sc, _edge_sc = _sc_kernels()

    degp = _deg_sc(dst3)
    hs1, dinv = _tc_pre(x, W1, degp)
    acc1 = _edge_sc(src3, dst3, hs1)
    hs2 = _tc_mid(acc1, hs1, dinv, b1.reshape(1, D), W2)
    acc2 = _edge_sc(src3, dst3, hs2)
    hs3 = _tc_mid(acc2, hs2, dinv, b2.reshape(1, D), W3)
    acc3 = _edge_sc(src3, dst3, hs3)
    out = _tc_fin(acc3, hs3, dinv, b3.reshape(1, D),
                  batch.reshape(NFBLK, 1, FBLK), batch.reshape(N_NODES, 1),
                  Wl, bl.reshape(1, N_CLASSES))
    return out
